# Initial kernel scaffold; baseline (speedup 1.0000x reference)
import jax
import jax.numpy as jnp
from jax.experimental import pallas as pl


def kernel(x, table):
    raise NotImplementedError("write your pallas kernel here")


# speedup vs baseline: 3.3700x; 3.3700x over previous
import functools

import jax
import jax.numpy as jnp
from jax import lax
from jax.experimental import pallas as pl
from jax.experimental.pallas import tpu as pltpu
from jax.experimental.pallas import tpu_sc as plsc

MAX_SEQ_LEN = 200
D_MODEL = 64

NUM_CORES = 2
NUM_SUBCORES = 16
NUM_WORKERS = NUM_CORES * NUM_SUBCORES

SEQS_PER_CHUNK = 2
CHUNK = SEQS_PER_CHUNK * MAX_SEQ_LEN
SUB = 80
NSUB = CHUNK // SUB


def _pos_encoding(seq_len, d_model):
    pos = jnp.arange(seq_len, dtype=jnp.float32)[:, None]
    exp = jnp.arange(0, d_model, 2, dtype=jnp.float32)
    stop = d_model // 2
    pe = jnp.zeros((seq_len, d_model), dtype=jnp.float32)
    pe = pe.at[:, 0::2].set(jnp.sin(pos / 10000 ** (exp / d_model)))
    pe = pe.at[:, 1::2].set(jnp.cos(pos / 10000 ** (exp[:stop] / d_model)))
    return pe


@jax.jit
def _embed(xf, table, pe):
    total = xf.shape[0]
    per_w = total // NUM_WORKERS
    n_chunks = per_w // CHUNK
    mesh = plsc.VectorSubcoreMesh(core_axis_name="c", subcore_axis_name="s")

    @functools.partial(
        pl.kernel,
        mesh=mesh,
        out_type=jax.ShapeDtypeStruct((total, D_MODEL), jnp.float32),
        scratch_types=[
            pltpu.VMEM((CHUNK,), jnp.int32),
            pltpu.VMEM((CHUNK, D_MODEL), jnp.float32),
            pltpu.VMEM((MAX_SEQ_LEN, D_MODEL), jnp.float32),
            pltpu.SemaphoreType.DMA,
        ],
        compiler_params=pltpu.CompilerParams(use_tc_tiling_on_sc=False),
    )
    def k(xf_hbm, tab_hbm, pe_hbm, out_hbm, idx_v, rows_v, pe_v, sem):
        wid = lax.axis_index("s") * NUM_CORES + lax.axis_index("c")
        base = wid * per_w
        pltpu.sync_copy(pe_hbm, pe_v)

        def chunk_body(c, carry):
            off = base + c * CHUNK
            pltpu.sync_copy(xf_hbm.at[pl.ds(off, CHUNK)], idx_v)
            copies = [
                pltpu.async_copy(
                    tab_hbm.at[idx_v.at[pl.ds(j * SUB, SUB)]],
                    rows_v.at[pl.ds(j * SUB, SUB)],
                    sem,
                )
                for j in range(NSUB)
            ]
            for cp in copies:
                cp.wait()

            def pos_body(p, carry2):
                for s in range(SEQS_PER_CHUNK):
                    for d in range(D_MODEL // 16):
                        plsc.addupdate(
                            rows_v.at[s * MAX_SEQ_LEN + p, pl.ds(d * 16, 16)],
                            pe_v[p, pl.ds(d * 16, 16)],
                        )
                return carry2

            lax.fori_loop(0, MAX_SEQ_LEN, pos_body, 0)
            pltpu.sync_copy(rows_v, out_hbm.at[pl.ds(off, CHUNK)])
            return carry

        lax.fori_loop(0, n_chunks, chunk_body, 0)

    return k(xf, table, pe)


def kernel(x, table):
    batch, seq_len = x.shape
    pe = _pos_encoding(seq_len, D_MODEL)
    xf = x.reshape(-1).astype(jnp.int32)
    out = _embed(xf, table, pe)
    return out.reshape(batch, seq_len, D_MODEL)


# speedup vs baseline: 3.7064x; 1.0998x over previous
import functools

import jax
import jax.numpy as jnp
from jax import lax
from jax.experimental import pallas as pl
from jax.experimental.pallas import tpu as pltpu
from jax.experimental.pallas import tpu_sc as plsc

MAX_SEQ_LEN = 200
D_MODEL = 64

NUM_CORES = 2
NUM_SUBCORES = 16
NUM_WORKERS = NUM_CORES * NUM_SUBCORES

SEQS_PER_CHUNK = 2
CHUNK = SEQS_PER_CHUNK * MAX_SEQ_LEN
SUB = 80
NSUB = CHUNK // SUB
NBUF = 2


def _pos_encoding(seq_len, d_model):
    pos = jnp.arange(seq_len, dtype=jnp.float32)[:, None]
    exp = jnp.arange(0, d_model, 2, dtype=jnp.float32)
    stop = d_model // 2
    pe = jnp.zeros((seq_len, d_model), jnp.float32)
    pe = pe.at[:, 0::2].set(jnp.sin(pos / 10000 ** (exp / d_model)))
    pe = pe.at[:, 1::2].set(jnp.cos(pos / 10000 ** (exp[:stop] / d_model)))
    return pe


@jax.jit
def _embed(xf, table, pe):
    total = xf.shape[0]
    per_w = total // NUM_WORKERS
    n_chunks = per_w // CHUNK
    n_outer = n_chunks // NBUF
    mesh = plsc.VectorSubcoreMesh(core_axis_name="c", subcore_axis_name="s")

    @functools.partial(
        pl.kernel,
        mesh=mesh,
        out_type=jax.ShapeDtypeStruct((total, D_MODEL), jnp.float32),
        scratch_types=[
            pltpu.VMEM((per_w,), jnp.int32),
            pltpu.VMEM((CHUNK, D_MODEL), jnp.float32),
            pltpu.VMEM((CHUNK, D_MODEL), jnp.float32),
            pltpu.VMEM((MAX_SEQ_LEN, D_MODEL), jnp.float32),
            pltpu.SemaphoreType.DMA,
            pltpu.SemaphoreType.DMA,
            pltpu.SemaphoreType.DMA,
            pltpu.SemaphoreType.DMA,
        ],
        compiler_params=pltpu.CompilerParams(use_tc_tiling_on_sc=False),
    )
    def k(xf_hbm, tab_hbm, pe_hbm, out_hbm,
          idx_all, rows0, rows1, pe_v, sg0, sg1, so0, so1):
        wid = lax.axis_index("s") * NUM_CORES + lax.axis_index("c")
        base = wid * per_w
        rows = [rows0, rows1]
        sg = [sg0, sg1]
        so = [so0, so1]

        pltpu.sync_copy(pe_hbm, pe_v)
        pltpu.sync_copy(xf_hbm.at[pl.ds(base, per_w)], idx_all)

        def fire_gather(g, b):
            off = g * CHUNK
            for j in range(NSUB):
                pltpu.async_copy(
                    tab_hbm.at[idx_all.at[pl.ds(off + j * SUB, SUB)]],
                    rows[b].at[pl.ds(j * SUB, SUB)],
                    sg[b],
                )

        def wait_gather(b):
            for j in range(NSUB):
                pltpu.make_async_copy(
                    tab_hbm.at[idx_all.at[pl.ds(j * SUB, SUB)]],
                    rows[b].at[pl.ds(j * SUB, SUB)],
                    sg[b],
                ).wait()

        def fire_out(g, b):
            pltpu.async_copy(
                rows[b], out_hbm.at[pl.ds(base + g * CHUNK, CHUNK)], so[b]
            )

        def wait_out(b):
            pltpu.make_async_copy(
                rows[b], out_hbm.at[pl.ds(base, CHUNK)], so[b]
            ).wait()

        def add_pe(b):
            def pos_body(p, carry):
                for s in range(SEQS_PER_CHUNK):
                    for d in range(D_MODEL // 16):
                        plsc.addupdate(
                            rows[b].at[s * MAX_SEQ_LEN + p, pl.ds(d * 16, 16)],
                            pe_v[p, pl.ds(d * 16, 16)],
                        )
                return carry

            lax.fori_loop(0, MAX_SEQ_LEN, pos_body, 0)

        fire_gather(0, 0)

        def outer(go, carry):
            for b in range(NBUF):
                g = go * NBUF + b
                nb = (b + 1) % NBUF
                wait_gather(b)
                add_pe(b)
                fire_out(g, b)
                if b == NBUF - 1:
                    @pl.when(go < n_outer - 1)
                    def _():
                        wait_out(nb)
                        fire_gather(g + 1, nb)
                else:
                    @pl.when(g > 0)
                    def _():
                        wait_out(nb)
                    fire_gather(g + 1, nb)
            return carry

        lax.fori_loop(0, n_outer, outer, 0)
        wait_out(0)
        wait_out(1)

    return k(xf, table, pe)


def kernel(x, table):
    batch, seq_len = x.shape
    pe = _pos_encoding(seq_len, D_MODEL)
    xf = x.reshape(-1).astype(jnp.int32)
    out = _embed(xf, table, pe)
    return out.reshape(batch, seq_len, D_MODEL)
